# baseline (speedup 1.0000x reference)
import functools

import jax
import jax.numpy as jnp
from jax import lax
from jax.experimental import pallas as pl
from jax.experimental.pallas import tpu as pltpu

N = 50000
E = 800000
DE = 32
DN = 32
H = 64

EDGE_BLOCK = 3200
NODE_BLOCK = 2000


def _layer_norm(h, g, beta):
    mu = jnp.mean(h, axis=-1, keepdims=True)
    var = jnp.mean((h - mu) ** 2, axis=-1, keepdims=True)
    return (h - mu) * lax.rsqrt(var + 1e-5) * g + beta


def _edge_mlp_body(x_ref, w1, b1, w2, b2, w3, b3, w4, b4, g, beta, out_ref):
    x = x_ref[...]
    h = jnp.maximum(jnp.dot(x, w1[...], preferred_element_type=jnp.float32) + b1[...], 0.0)
    h = jnp.maximum(jnp.dot(h, w2[...], preferred_element_type=jnp.float32) + b2[...], 0.0)
    h = jnp.maximum(jnp.dot(h, w3[...], preferred_element_type=jnp.float32) + b3[...], 0.0)
    h = jnp.dot(h, w4[...], preferred_element_type=jnp.float32) + b4[...]
    out_ref[...] = _layer_norm(h, g[...], beta[...])


def _edge_mlp(x_edge, p0):
    W1, b1, W2, b2, W3, b3, W4, b4, g, beta = p0
    small = [W1, b1.reshape(1, -1), W2, b2.reshape(1, -1), W3, b3.reshape(1, -1),
             W4, b4.reshape(1, -1), g.reshape(1, -1), beta.reshape(1, -1)]
    grid = (E // EDGE_BLOCK,)
    full = lambda a: pl.BlockSpec(a.shape, lambda i: (0, 0))
    return pl.pallas_call(
        _edge_mlp_body,
        grid=grid,
        in_specs=[pl.BlockSpec((EDGE_BLOCK, DE), lambda i: (i, 0))] + [full(a) for a in small],
        out_specs=pl.BlockSpec((EDGE_BLOCK, H), lambda i: (i, 0)),
        out_shape=jax.ShapeDtypeStruct((E, H), jnp.float32),
    )(x_edge, *small)


def _node_mlp_body(xn_ref, xa_ref, w1a, w1b, b1, w2, b2, w3, b3, w4, b4, g, beta, out_ref):
    h = (jnp.dot(xn_ref[...], w1a[...], preferred_element_type=jnp.float32)
         + jnp.dot(xa_ref[...], w1b[...], preferred_element_type=jnp.float32) + b1[...])
    h = jnp.maximum(h, 0.0)
    h = jnp.maximum(jnp.dot(h, w2[...], preferred_element_type=jnp.float32) + b2[...], 0.0)
    h = jnp.maximum(jnp.dot(h, w3[...], preferred_element_type=jnp.float32) + b3[...], 0.0)
    h = jnp.dot(h, w4[...], preferred_element_type=jnp.float32) + b4[...]
    out_ref[...] = _layer_norm(h, g[...], beta[...])


def _node_mlp(x_node, x_agg, p1):
    W1, b1, W2, b2, W3, b3, W4, b4, g, beta = p1
    small = [W1[:DN], W1[DN:], b1.reshape(1, -1), W2, b2.reshape(1, -1), W3,
             b3.reshape(1, -1), W4, b4.reshape(1, -1), g.reshape(1, -1), beta.reshape(1, -1)]
    grid = (N // NODE_BLOCK,)
    full = lambda a: pl.BlockSpec(a.shape, lambda i: (0, 0))
    return pl.pallas_call(
        _node_mlp_body,
        grid=grid,
        in_specs=[pl.BlockSpec((NODE_BLOCK, DN), lambda i: (i, 0)),
                  pl.BlockSpec((NODE_BLOCK, H), lambda i: (i, 0))] + [full(a) for a in small],
        out_specs=pl.BlockSpec((NODE_BLOCK, H), lambda i: (i, 0)),
        out_shape=jax.ShapeDtypeStruct((N, H), jnp.float32),
    )(x_node, x_agg, *small)


def kernel(x_node, x_edge, edge_index, p0, p1):
    x_edge1 = _edge_mlp(x_edge, p0)
    x_agg = jax.ops.segment_sum(x_edge1, edge_index[:, 0], num_segments=N)
    x_agg = x_agg + jax.ops.segment_sum(x_edge1, edge_index[:, 1], num_segments=N)
    x_node3 = _node_mlp(x_node, x_agg, p1)
    return (x_node3, x_edge1)


# speedup vs baseline: 1.7835x; 1.7835x over previous
import functools

import jax
import jax.numpy as jnp
from jax import lax
from jax.experimental import pallas as pl
from jax.experimental.pallas import tpu as pltpu
from jax.experimental.pallas import tpu_sc as plsc

N = 50000
E = 800000
DE = 32
DN = 32
H = 64

EDGE_BLOCK = 3200
NODE_BLOCK = 2000


def _layer_norm(h, g, beta):
    mu = jnp.mean(h, axis=-1, keepdims=True)
    var = jnp.mean((h - mu) ** 2, axis=-1, keepdims=True)
    return (h - mu) * lax.rsqrt(var + 1e-5) * g + beta


def _edge_mlp_body(x_ref, w1, b1, w2, b2, w3, b3, w4, b4, g, beta, out_ref):
    x = x_ref[...]
    h = jnp.maximum(jnp.dot(x, w1[...], preferred_element_type=jnp.float32) + b1[...], 0.0)
    h = jnp.maximum(jnp.dot(h, w2[...], preferred_element_type=jnp.float32) + b2[...], 0.0)
    h = jnp.maximum(jnp.dot(h, w3[...], preferred_element_type=jnp.float32) + b3[...], 0.0)
    h = jnp.dot(h, w4[...], preferred_element_type=jnp.float32) + b4[...]
    out_ref[...] = _layer_norm(h, g[...], beta[...])


def _edge_mlp(x_edge, p0):
    W1, b1, W2, b2, W3, b3, W4, b4, g, beta = p0
    small = [W1, b1.reshape(1, -1), W2, b2.reshape(1, -1), W3, b3.reshape(1, -1),
             W4, b4.reshape(1, -1), g.reshape(1, -1), beta.reshape(1, -1)]
    grid = (E // EDGE_BLOCK,)
    full = lambda a: pl.BlockSpec(a.shape, lambda i: (0, 0))
    return pl.pallas_call(
        _edge_mlp_body,
        grid=grid,
        in_specs=[pl.BlockSpec((EDGE_BLOCK, DE), lambda i: (i, 0))] + [full(a) for a in small],
        out_specs=pl.BlockSpec((EDGE_BLOCK, H), lambda i: (i, 0)),
        out_shape=jax.ShapeDtypeStruct((E, H), jnp.float32),
    )(x_edge, *small)


def _node_mlp_body(xn_ref, xa_ref, w1a, w1b, b1, w2, b2, w3, b3, w4, b4, g, beta, out_ref):
    h = (jnp.dot(xn_ref[...], w1a[...], preferred_element_type=jnp.float32)
         + jnp.dot(xa_ref[...], w1b[...], preferred_element_type=jnp.float32) + b1[...])
    h = jnp.maximum(h, 0.0)
    h = jnp.maximum(jnp.dot(h, w2[...], preferred_element_type=jnp.float32) + b2[...], 0.0)
    h = jnp.maximum(jnp.dot(h, w3[...], preferred_element_type=jnp.float32) + b3[...], 0.0)
    h = jnp.dot(h, w4[...], preferred_element_type=jnp.float32) + b4[...]
    out_ref[...] = _layer_norm(h, g[...], beta[...])


def _node_mlp(x_node, x_agg, p1):
    W1, b1, W2, b2, W3, b3, W4, b4, g, beta = p1
    small = [W1[:DN], W1[DN:], b1.reshape(1, -1), W2, b2.reshape(1, -1), W3,
             b3.reshape(1, -1), W4, b4.reshape(1, -1), g.reshape(1, -1), beta.reshape(1, -1)]
    grid = (N // NODE_BLOCK,)
    full = lambda a: pl.BlockSpec(a.shape, lambda i: (0, 0))
    return pl.pallas_call(
        _node_mlp_body,
        grid=grid,
        in_specs=[pl.BlockSpec((NODE_BLOCK, DN), lambda i: (i, 0)),
                  pl.BlockSpec((NODE_BLOCK, H), lambda i: (i, 0))] + [full(a) for a in small],
        out_specs=pl.BlockSpec((NODE_BLOCK, H), lambda i: (i, 0)),
        out_shape=jax.ShapeDtypeStruct((N, H), jnp.float32),
    )(x_node, x_agg, *small)


HALF = N // 2
ACC_ROWS = 25088
TRASH = HALF
C = 80
CHUNKS = (E // 16) // C
ZROWS = 112
ZREP = 14
OUT_PER_TILE = 1560


def _sc_scatter(x_edge1, src_idx, dst_idx):
    mesh = plsc.VectorSubcoreMesh(core_axis_name="c", subcore_axis_name="s",
                                  num_cores=2, num_subcores=16)

    @functools.partial(
        pl.kernel,
        out_type=jax.ShapeDtypeStruct((N, H), jnp.float32),
        mesh=mesh,
        compiler_params=pltpu.CompilerParams(use_tc_tiling_on_sc=False),
        scratch_types=[
            pltpu.VMEM_SHARED((ACC_ROWS, H), jnp.float32),
            pltpu.VMEM((C, H), jnp.float32),
            pltpu.VMEM((C,), jnp.int32),
            pltpu.VMEM((C,), jnp.int32),
            pltpu.VMEM((ZROWS, H), jnp.float32),
        ],
    )
    def scatter_kernel(xe_hbm, src_hbm, dst_hbm, out_hbm, acc_sh, rows_v, src_v, dst_v, zbuf):
        c = lax.axis_index("c")
        s = lax.axis_index("s")
        z16 = jnp.zeros((16,), jnp.float32)

        def zero_zbuf(i, carry):
            for j in range(H // 16):
                zbuf[i, pl.ds(j * 16, 16)] = z16
            return carry
        lax.fori_loop(0, ZROWS, zero_zbuf, 0)

        def zero_acc(k, carry):
            pltpu.sync_copy(zbuf, acc_sh.at[pl.ds(s * (ZROWS * ZREP) + k * ZROWS, ZROWS)])
            return carry
        lax.fori_loop(0, ZREP, zero_acc, 0)
        plsc.subcore_barrier()

        lo = c * HALF

        def chunk_body(i, carry):
            base = s * (C * CHUNKS) + i * C
            pltpu.sync_copy(src_hbm.at[pl.ds(base, C)], src_v)
            pltpu.sync_copy(dst_hbm.at[pl.ds(base, C)], dst_v)
            pltpu.sync_copy(xe_hbm.at[pl.ds(base, C)], rows_v)
            for g in range(C // 16):
                sv = src_v[pl.ds(g * 16, 16)] - lo
                src_v[pl.ds(g * 16, 16)] = jnp.where((sv >= 0) & (sv < HALF), sv, TRASH)
                dv = dst_v[pl.ds(g * 16, 16)] - lo
                dst_v[pl.ds(g * 16, 16)] = jnp.where((dv >= 0) & (dv < HALF), dv, TRASH)
            pltpu.sync_copy(rows_v, acc_sh.at[src_v], add=True)
            pltpu.sync_copy(rows_v, acc_sh.at[dst_v], add=True)
            return carry
        lax.fori_loop(0, CHUNKS, chunk_body, 0)
        plsc.subcore_barrier()

        pltpu.sync_copy(acc_sh.at[pl.ds(s * OUT_PER_TILE, OUT_PER_TILE)],
                        out_hbm.at[pl.ds(c * HALF + s * OUT_PER_TILE, OUT_PER_TILE)])

        @pl.when(s == 0)
        def _copy_tail():
            pltpu.sync_copy(acc_sh.at[pl.ds(16 * OUT_PER_TILE, HALF - 16 * OUT_PER_TILE)],
                            out_hbm.at[pl.ds(c * HALF + 16 * OUT_PER_TILE,
                                             HALF - 16 * OUT_PER_TILE)])

    return scatter_kernel(x_edge1, src_idx, dst_idx)


def kernel(x_node, x_edge, edge_index, p0, p1):
    x_edge1 = _edge_mlp(x_edge, p0)
    x_agg = _sc_scatter(x_edge1, edge_index[:, 0], edge_index[:, 1])
    x_node3 = _node_mlp(x_node, x_agg, p1)
    return (x_node3, x_edge1)


# speedup vs baseline: 2.1683x; 1.2157x over previous
import functools

import jax
import jax.numpy as jnp
from jax import lax
from jax.experimental import pallas as pl
from jax.experimental.pallas import tpu as pltpu
from jax.experimental.pallas import tpu_sc as plsc

N = 50000
E = 800000
DE = 32
DN = 32
H = 64

EDGE_BLOCK = 3200
NODE_BLOCK = 2000


def _layer_norm(h, g, beta):
    mu = jnp.mean(h, axis=-1, keepdims=True)
    var = jnp.mean((h - mu) ** 2, axis=-1, keepdims=True)
    return (h - mu) * lax.rsqrt(var + 1e-5) * g + beta


def _edge_mlp_body(x_ref, w1, b1, w2, b2, w3, b3, w4, b4, g, beta, out_ref):
    x = x_ref[...]
    h = jnp.maximum(jnp.dot(x, w1[...], preferred_element_type=jnp.float32) + b1[...], 0.0)
    h = jnp.maximum(jnp.dot(h, w2[...], preferred_element_type=jnp.float32) + b2[...], 0.0)
    h = jnp.maximum(jnp.dot(h, w3[...], preferred_element_type=jnp.float32) + b3[...], 0.0)
    h = jnp.dot(h, w4[...], preferred_element_type=jnp.float32) + b4[...]
    out_ref[...] = _layer_norm(h, g[...], beta[...])


def _edge_mlp(x_edge, p0):
    W1, b1, W2, b2, W3, b3, W4, b4, g, beta = p0
    small = [W1, b1.reshape(1, -1), W2, b2.reshape(1, -1), W3, b3.reshape(1, -1),
             W4, b4.reshape(1, -1), g.reshape(1, -1), beta.reshape(1, -1)]
    grid = (E // EDGE_BLOCK,)
    full = lambda a: pl.BlockSpec(a.shape, lambda i: (0, 0))
    return pl.pallas_call(
        _edge_mlp_body,
        grid=grid,
        in_specs=[pl.BlockSpec((EDGE_BLOCK, DE), lambda i: (i, 0))] + [full(a) for a in small],
        out_specs=pl.BlockSpec((EDGE_BLOCK, H), lambda i: (i, 0)),
        out_shape=jax.ShapeDtypeStruct((E, H), jnp.float32),
    )(x_edge, *small)


def _node_mlp_body(xn_ref, xa_ref, w1a, w1b, b1, w2, b2, w3, b3, w4, b4, g, beta, out_ref):
    h = (jnp.dot(xn_ref[...], w1a[...], preferred_element_type=jnp.float32)
         + jnp.dot(xa_ref[...], w1b[...], preferred_element_type=jnp.float32) + b1[...])
    h = jnp.maximum(h, 0.0)
    h = jnp.maximum(jnp.dot(h, w2[...], preferred_element_type=jnp.float32) + b2[...], 0.0)
    h = jnp.maximum(jnp.dot(h, w3[...], preferred_element_type=jnp.float32) + b3[...], 0.0)
    h = jnp.dot(h, w4[...], preferred_element_type=jnp.float32) + b4[...]
    out_ref[...] = _layer_norm(h, g[...], beta[...])


def _node_mlp(x_node, x_agg, p1):
    W1, b1, W2, b2, W3, b3, W4, b4, g, beta = p1
    small = [W1[:DN], W1[DN:], b1.reshape(1, -1), W2, b2.reshape(1, -1), W3,
             b3.reshape(1, -1), W4, b4.reshape(1, -1), g.reshape(1, -1), beta.reshape(1, -1)]
    grid = (N // NODE_BLOCK,)
    full = lambda a: pl.BlockSpec(a.shape, lambda i: (0, 0))
    return pl.pallas_call(
        _node_mlp_body,
        grid=grid,
        in_specs=[pl.BlockSpec((NODE_BLOCK, DN), lambda i: (i, 0)),
                  pl.BlockSpec((NODE_BLOCK, H), lambda i: (i, 0))] + [full(a) for a in small],
        out_specs=pl.BlockSpec((NODE_BLOCK, H), lambda i: (i, 0)),
        out_shape=jax.ShapeDtypeStruct((N, H), jnp.float32),
    )(x_node, x_agg, *small)


HALF = N // 2
ACC_ROWS = 25088
TRASH = HALF
C = 80
CHUNKS = (E // 16) // C
NBUF = 4
MAIN = (CHUNKS // NBUF) * NBUF
ZROWS = 112
ZREP = 14
OUT_PER_TILE = 1560


def _sc_scatter(x_edge1, src_idx, dst_idx):
    mesh = plsc.VectorSubcoreMesh(core_axis_name="c", subcore_axis_name="s",
                                  num_cores=2, num_subcores=16)

    @functools.partial(
        pl.kernel,
        out_type=jax.ShapeDtypeStruct((N, H), jnp.float32),
        mesh=mesh,
        compiler_params=pltpu.CompilerParams(use_tc_tiling_on_sc=False),
        scratch_types=[
            pltpu.VMEM_SHARED((ACC_ROWS, H), jnp.float32),
            [pltpu.VMEM((C, H), jnp.float32)] * NBUF,
            [pltpu.VMEM((C,), jnp.int32)] * NBUF,
            [pltpu.VMEM((C,), jnp.int32)] * NBUF,
            pltpu.VMEM((ZROWS, H), jnp.float32),
            [pltpu.SemaphoreType.DMA] * NBUF,
            [pltpu.SemaphoreType.DMA] * NBUF,
        ],
    )
    def scatter_kernel(xe_hbm, src_hbm, dst_hbm, out_hbm, acc_sh, rows_v, src_v, dst_v, zbuf,
                       load_sem, scat_sem):
        c = lax.axis_index("c")
        s = lax.axis_index("s")
        z16 = jnp.zeros((16,), jnp.float32)

        def zero_zbuf(i, carry):
            for j in range(H // 16):
                zbuf[i, pl.ds(j * 16, 16)] = z16
            return carry
        lax.fori_loop(0, ZROWS, zero_zbuf, 0)

        def zero_acc(k, carry):
            pltpu.sync_copy(zbuf, acc_sh.at[pl.ds(s * (ZROWS * ZREP) + k * ZROWS, ZROWS)])
            return carry
        lax.fori_loop(0, ZREP, zero_acc, 0)
        plsc.subcore_barrier()

        lo = c * HALF
        tile_base = s * (C * CHUNKS)

        def clamp(buf):
            for g in range(C // 16):
                v = buf[pl.ds(g * 16, 16)] - lo
                buf[pl.ds(g * 16, 16)] = jnp.where((v >= 0) & (v < HALF), v, TRASH)

        def issue_loads(k, b):
            base = tile_base + k * C
            pltpu.async_copy(xe_hbm.at[pl.ds(base, C)], rows_v[b], load_sem[b])
            pltpu.async_copy(src_hbm.at[pl.ds(base, C)], src_v[b], load_sem[b])
            pltpu.async_copy(dst_hbm.at[pl.ds(base, C)], dst_v[b], load_sem[b])

        def drain_loads(b):
            pltpu.make_async_copy(xe_hbm.at[pl.ds(0, C)], rows_v[b], load_sem[b]).wait()
            pltpu.make_async_copy(src_hbm.at[pl.ds(0, C)], src_v[b], load_sem[b]).wait()
            pltpu.make_async_copy(dst_hbm.at[pl.ds(0, C)], dst_v[b], load_sem[b]).wait()

        def drain_scats(b):
            pltpu.make_async_copy(rows_v[b], acc_sh.at[src_v[b]], scat_sem[b]).wait()
            pltpu.make_async_copy(rows_v[b], acc_sh.at[dst_v[b]], scat_sem[b]).wait()

        issue_loads(0, 0)
        issue_loads(1, 1)

        def outer_body(t, carry):
            for b in range(NBUF):
                k = t * NBUF + b
                nb = (b + 2) % NBUF

                @pl.when(k >= 2)
                def _drain():
                    drain_scats(nb)

                @pl.when(k + 2 < MAIN)
                def _prefetch():
                    issue_loads(k + 2, nb)

                drain_loads(b)
                clamp(src_v[b])
                clamp(dst_v[b])
                pltpu.async_copy(rows_v[b], acc_sh.at[src_v[b]], scat_sem[b], add=True)
                pltpu.async_copy(rows_v[b], acc_sh.at[dst_v[b]], scat_sem[b], add=True)
            return carry
        lax.fori_loop(0, MAIN // NBUF, outer_body, 0)
        drain_scats((MAIN - 2) % NBUF)
        drain_scats((MAIN - 1) % NBUF)

        for k in range(MAIN, CHUNKS):
            base = tile_base + k * C
            pltpu.sync_copy(src_hbm.at[pl.ds(base, C)], src_v[0])
            pltpu.sync_copy(dst_hbm.at[pl.ds(base, C)], dst_v[0])
            pltpu.sync_copy(xe_hbm.at[pl.ds(base, C)], rows_v[0])
            clamp(src_v[0])
            clamp(dst_v[0])
            pltpu.sync_copy(rows_v[0], acc_sh.at[src_v[0]], add=True)
            pltpu.sync_copy(rows_v[0], acc_sh.at[dst_v[0]], add=True)
        plsc.subcore_barrier()

        pltpu.sync_copy(acc_sh.at[pl.ds(s * OUT_PER_TILE, OUT_PER_TILE)],
                        out_hbm.at[pl.ds(c * HALF + s * OUT_PER_TILE, OUT_PER_TILE)])

        @pl.when(s == 0)
        def _copy_tail():
            pltpu.sync_copy(acc_sh.at[pl.ds(16 * OUT_PER_TILE, HALF - 16 * OUT_PER_TILE)],
                            out_hbm.at[pl.ds(c * HALF + 16 * OUT_PER_TILE,
                                             HALF - 16 * OUT_PER_TILE)])

    return scatter_kernel(x_edge1, src_idx, dst_idx)


def kernel(x_node, x_edge, edge_index, p0, p1):
    x_edge1 = _edge_mlp(x_edge, p0)
    x_agg = _sc_scatter(x_edge1, edge_index[:, 0], edge_index[:, 1])
    x_node3 = _node_mlp(x_node, x_agg, p1)
    return (x_node3, x_edge1)


# speedup vs baseline: 3.0934x; 1.4267x over previous
import functools

import jax
import jax.numpy as jnp
from jax import lax
from jax.experimental import pallas as pl
from jax.experimental.pallas import tpu as pltpu
from jax.experimental.pallas import tpu_sc as plsc

N = 50000
E = 800000
DE = 32
DN = 32
H = 64

EDGE_BLOCK = 3200
N_PAD = 50048
NODE_BLOCK = 2176


def _bf(x):
    return x.astype(jnp.bfloat16)


def _layer_norm_t(h, g, beta):
    mu = jnp.mean(h, axis=0, keepdims=True)
    var = jnp.mean((h - mu) ** 2, axis=0, keepdims=True)
    return (h - mu) * lax.rsqrt(var + 1e-5) * g + beta


def _edge_mlp_body(xt_ref, w1, b1, w2, b2, w3, b3, w4, b4, g, beta, outt_ref, packed_ref):
    xt = xt_ref[...]
    h = jnp.maximum(jnp.dot(_bf(w1[...]), _bf(xt), preferred_element_type=jnp.float32) + b1[...], 0.0)
    h = jnp.maximum(jnp.dot(_bf(w2[...]), _bf(h), preferred_element_type=jnp.float32) + b2[...], 0.0)
    h = jnp.maximum(jnp.dot(_bf(w3[...]), _bf(h), preferred_element_type=jnp.float32) + b3[...], 0.0)
    h = jnp.dot(_bf(w4[...]), _bf(h), preferred_element_type=jnp.float32) + b4[...]
    out = _layer_norm_t(h, g[...], beta[...])
    outt_ref[...] = out
    packed_ref[:, pl.ds(0, H)] = out.T


def _edge_mlp(x_edge, p0):
    W1, b1, W2, b2, W3, b3, W4, b4, g, beta = p0
    small = [W1.T, b1.reshape(-1, 1), W2.T, b2.reshape(-1, 1), W3.T, b3.reshape(-1, 1),
             W4.T, b4.reshape(-1, 1), g.reshape(-1, 1), beta.reshape(-1, 1)]
    grid = (E // EDGE_BLOCK,)
    full = lambda a: pl.BlockSpec(a.shape, lambda i: (0, 0))
    return pl.pallas_call(
        _edge_mlp_body,
        grid=grid,
        in_specs=[pl.BlockSpec((DE, EDGE_BLOCK), lambda i: (0, i))] + [full(a) for a in small],
        out_specs=[pl.BlockSpec((H, EDGE_BLOCK), lambda i: (0, i)),
                   pl.BlockSpec((EDGE_BLOCK, 2 * H), lambda i: (i, 0))],
        out_shape=[jax.ShapeDtypeStruct((H, E), jnp.float32),
                   jax.ShapeDtypeStruct((E, 2 * H), jnp.float32)],
    )(x_edge.T, *small)


def _node_mlp_body(xnt_ref, xa_ref, w1a, w1b, b1, w2, b2, w3, b3, w4, b4, g, beta, outt_ref):
    xat = xa_ref[...].T
    h = (jnp.dot(w1a[...], xnt_ref[...], preferred_element_type=jnp.float32)
         + jnp.dot(w1b[...], xat, preferred_element_type=jnp.float32) + b1[...])
    h = jnp.maximum(h, 0.0)
    h = jnp.maximum(jnp.dot(w2[...], h, preferred_element_type=jnp.float32) + b2[...], 0.0)
    h = jnp.maximum(jnp.dot(w3[...], h, preferred_element_type=jnp.float32) + b3[...], 0.0)
    h = jnp.dot(w4[...], h, preferred_element_type=jnp.float32) + b4[...]
    outt_ref[...] = _layer_norm_t(h, g[...], beta[...])


def _node_mlp(x_node, x_agg, p1):
    W1, b1, W2, b2, W3, b3, W4, b4, g, beta = p1
    small = [W1[:DN].T, W1[DN:].T, b1.reshape(-1, 1), W2.T, b2.reshape(-1, 1), W3.T,
             b3.reshape(-1, 1), W4.T, b4.reshape(-1, 1), g.reshape(-1, 1), beta.reshape(-1, 1)]
    grid = (N_PAD // NODE_BLOCK,)
    full = lambda a: pl.BlockSpec(a.shape, lambda i: (0, 0))
    xnt = jnp.pad(x_node.T, ((0, 0), (0, N_PAD - N)))
    return pl.pallas_call(
        _node_mlp_body,
        grid=grid,
        in_specs=[pl.BlockSpec((DN, NODE_BLOCK), lambda i: (0, i)),
                  pl.BlockSpec((NODE_BLOCK, H), lambda i: (i, 0))] + [full(a) for a in small],
        out_specs=pl.BlockSpec((H, NODE_BLOCK), lambda i: (0, i)),
        out_shape=jax.ShapeDtypeStruct((H, N_PAD), jnp.float32),
    )(xnt, x_agg, *small)


HALF = N // 2
ACC_ROWS = 25088
TRASH = HALF
C = 80
CHUNKS = (E // 16) // C
NBUF = 4
MAIN = (CHUNKS // NBUF) * NBUF
ZROWS = 112
ZREP = 14
OUT_PER_TILE = 1560


def _sc_scatter(x_edge1, src_idx, dst_idx):
    mesh = plsc.VectorSubcoreMesh(core_axis_name="c", subcore_axis_name="s",
                                  num_cores=2, num_subcores=16)

    @functools.partial(
        pl.kernel,
        out_type=jax.ShapeDtypeStruct((N_PAD, H), jnp.float32),
        mesh=mesh,
        compiler_params=pltpu.CompilerParams(use_tc_tiling_on_sc=False),
        scratch_types=[
            pltpu.VMEM_SHARED((ACC_ROWS, H), jnp.float32),
            [pltpu.VMEM((C, H), jnp.float32)] * NBUF,
            [pltpu.VMEM((C,), jnp.int32)] * NBUF,
            [pltpu.VMEM((C,), jnp.int32)] * NBUF,
            pltpu.VMEM((ZROWS, H), jnp.float32),
            [pltpu.SemaphoreType.DMA] * NBUF,
            [pltpu.SemaphoreType.DMA] * NBUF,
        ],
    )
    def scatter_kernel(xe_hbm, src_hbm, dst_hbm, out_hbm, acc_sh, rows_v, src_v, dst_v, zbuf,
                       load_sem, scat_sem):
        c = lax.axis_index("c")
        s = lax.axis_index("s")
        z16 = jnp.zeros((16,), jnp.float32)

        def zero_zbuf(i, carry):
            for j in range(H // 16):
                zbuf[i, pl.ds(j * 16, 16)] = z16
            return carry
        lax.fori_loop(0, ZROWS, zero_zbuf, 0)

        def zero_acc(k, carry):
            pltpu.sync_copy(zbuf, acc_sh.at[pl.ds(s * (ZROWS * ZREP) + k * ZROWS, ZROWS)])
            return carry
        lax.fori_loop(0, ZREP, zero_acc, 0)
        plsc.subcore_barrier()

        lo = c * HALF
        tile_base = s * (C * CHUNKS)

        def clamp(buf):
            for g in range(C // 16):
                v = buf[pl.ds(g * 16, 16)] - lo
                buf[pl.ds(g * 16, 16)] = jnp.where((v >= 0) & (v < HALF), v, TRASH)

        def issue_loads(k, b):
            base = tile_base + k * C
            pltpu.async_copy(xe_hbm.at[pl.ds(base, C), pl.ds(0, H)], rows_v[b], load_sem[b])
            pltpu.async_copy(src_hbm.at[pl.ds(base, C)], src_v[b], load_sem[b])
            pltpu.async_copy(dst_hbm.at[pl.ds(base, C)], dst_v[b], load_sem[b])

        def drain_loads(b):
            pltpu.make_async_copy(xe_hbm.at[pl.ds(0, C), pl.ds(0, H)], rows_v[b], load_sem[b]).wait()
            pltpu.make_async_copy(src_hbm.at[pl.ds(0, C)], src_v[b], load_sem[b]).wait()
            pltpu.make_async_copy(dst_hbm.at[pl.ds(0, C)], dst_v[b], load_sem[b]).wait()

        def drain_scats(b):
            pltpu.make_async_copy(rows_v[b], acc_sh.at[src_v[b]], scat_sem[b]).wait()
            pltpu.make_async_copy(rows_v[b], acc_sh.at[dst_v[b]], scat_sem[b]).wait()

        issue_loads(0, 0)
        issue_loads(1, 1)

        def outer_body(t, carry):
            for b in range(NBUF):
                k = t * NBUF + b
                nb = (b + 2) % NBUF

                @pl.when(k >= 2)
                def _drain():
                    drain_scats(nb)

                @pl.when(k + 2 < MAIN)
                def _prefetch():
                    issue_loads(k + 2, nb)

                drain_loads(b)
                clamp(src_v[b])
                clamp(dst_v[b])
                pltpu.async_copy(rows_v[b], acc_sh.at[src_v[b]], scat_sem[b], add=True)
                pltpu.async_copy(rows_v[b], acc_sh.at[dst_v[b]], scat_sem[b], add=True)
            return carry
        lax.fori_loop(0, MAIN // NBUF, outer_body, 0)
        drain_scats((MAIN - 2) % NBUF)
        drain_scats((MAIN - 1) % NBUF)

        for k in range(MAIN, CHUNKS):
            base = tile_base + k * C
            pltpu.sync_copy(src_hbm.at[pl.ds(base, C)], src_v[0])
            pltpu.sync_copy(dst_hbm.at[pl.ds(base, C)], dst_v[0])
            pltpu.sync_copy(xe_hbm.at[pl.ds(base, C), pl.ds(0, H)], rows_v[0])
            clamp(src_v[0])
            clamp(dst_v[0])
            pltpu.sync_copy(rows_v[0], acc_sh.at[src_v[0]], add=True)
            pltpu.sync_copy(rows_v[0], acc_sh.at[dst_v[0]], add=True)
        plsc.subcore_barrier()

        pltpu.sync_copy(acc_sh.at[pl.ds(s * OUT_PER_TILE, OUT_PER_TILE)],
                        out_hbm.at[pl.ds(c * HALF + s * OUT_PER_TILE, OUT_PER_TILE)])

        @pl.when(s == 0)
        def _copy_tail():
            pltpu.sync_copy(acc_sh.at[pl.ds(16 * OUT_PER_TILE, HALF - 16 * OUT_PER_TILE)],
                            out_hbm.at[pl.ds(c * HALF + 16 * OUT_PER_TILE,
                                             HALF - 16 * OUT_PER_TILE)])

    return scatter_kernel(x_edge1, src_idx, dst_idx)


def kernel(x_node, x_edge, edge_index, p0, p1):
    x_edge1_t, x_edge1_packed = _edge_mlp(x_edge, p0)
    x_agg = _sc_scatter(x_edge1_packed, edge_index[:, 0], edge_index[:, 1])
    x_node3_t = _node_mlp(x_node, x_agg, p1)
    return (x_node3_t[:, :N].T, x_edge1_t.T)


# speedup vs baseline: 3.0935x; 1.0000x over previous
import functools

import jax
import jax.numpy as jnp
from jax import lax
from jax.experimental import pallas as pl
from jax.experimental.pallas import tpu as pltpu
from jax.experimental.pallas import tpu_sc as plsc

N = 50000
E = 800000
DE = 32
DN = 32
H = 64

EDGE_BLOCK = 3200
N_PAD = 50048
NODE_BLOCK = 2176


def _bf(x):
    return x.astype(jnp.bfloat16)


def _layer_norm_t(h, g, beta):
    mu = jnp.mean(h, axis=0, keepdims=True)
    var = jnp.mean((h - mu) ** 2, axis=0, keepdims=True)
    return (h - mu) * lax.rsqrt(var + 1e-5) * g + beta


def _edge_mlp_body(xt_ref, w1, b1, w2, b2, w3, b3, w4, b4, g, beta, outt_ref, packed_ref):
    xt = xt_ref[...]
    h = jnp.maximum(jnp.dot(_bf(w1[...]), _bf(xt), preferred_element_type=jnp.float32) + b1[...], 0.0)
    h = jnp.maximum(jnp.dot(_bf(w2[...]), _bf(h), preferred_element_type=jnp.float32) + b2[...], 0.0)
    h = jnp.maximum(jnp.dot(_bf(w3[...]), _bf(h), preferred_element_type=jnp.float32) + b3[...], 0.0)
    h = jnp.dot(_bf(w4[...]), _bf(h), preferred_element_type=jnp.float32) + b4[...]
    out = _layer_norm_t(h, g[...], beta[...])
    outt_ref[...] = out
    packed_ref[:, pl.ds(0, H)] = out.T


def _edge_mlp(x_edge, p0):
    W1, b1, W2, b2, W3, b3, W4, b4, g, beta = p0
    small = [W1.T, b1.reshape(-1, 1), W2.T, b2.reshape(-1, 1), W3.T, b3.reshape(-1, 1),
             W4.T, b4.reshape(-1, 1), g.reshape(-1, 1), beta.reshape(-1, 1)]
    grid = (E // EDGE_BLOCK,)
    full = lambda a: pl.BlockSpec(a.shape, lambda i: (0, 0))
    return pl.pallas_call(
        _edge_mlp_body,
        grid=grid,
        in_specs=[pl.BlockSpec((DE, EDGE_BLOCK), lambda i: (0, i))] + [full(a) for a in small],
        out_specs=[pl.BlockSpec((H, EDGE_BLOCK), lambda i: (0, i)),
                   pl.BlockSpec((EDGE_BLOCK, 2 * H), lambda i: (i, 0))],
        out_shape=[jax.ShapeDtypeStruct((H, E), jnp.float32),
                   jax.ShapeDtypeStruct((E, 2 * H), jnp.float32)],
    )(x_edge.T, *small)


def _node_mlp_body(xnt_ref, xa_ref, w1a, w1b, b1, w2, b2, w3, b3, w4, b4, g, beta, outt_ref):
    xat = xa_ref[...].T
    h = (jnp.dot(w1a[...], xnt_ref[...], preferred_element_type=jnp.float32)
         + jnp.dot(w1b[...], xat, preferred_element_type=jnp.float32) + b1[...])
    h = jnp.maximum(h, 0.0)
    h = jnp.maximum(jnp.dot(w2[...], h, preferred_element_type=jnp.float32) + b2[...], 0.0)
    h = jnp.maximum(jnp.dot(w3[...], h, preferred_element_type=jnp.float32) + b3[...], 0.0)
    h = jnp.dot(w4[...], h, preferred_element_type=jnp.float32) + b4[...]
    outt_ref[...] = _layer_norm_t(h, g[...], beta[...])


def _node_mlp(x_node, x_agg, p1):
    W1, b1, W2, b2, W3, b3, W4, b4, g, beta = p1
    small = [W1[:DN].T, W1[DN:].T, b1.reshape(-1, 1), W2.T, b2.reshape(-1, 1), W3.T,
             b3.reshape(-1, 1), W4.T, b4.reshape(-1, 1), g.reshape(-1, 1), beta.reshape(-1, 1)]
    grid = (N_PAD // NODE_BLOCK,)
    full = lambda a: pl.BlockSpec(a.shape, lambda i: (0, 0))
    xnt = jnp.pad(x_node.T, ((0, 0), (0, N_PAD - N)))
    return pl.pallas_call(
        _node_mlp_body,
        grid=grid,
        in_specs=[pl.BlockSpec((DN, NODE_BLOCK), lambda i: (0, i)),
                  pl.BlockSpec((NODE_BLOCK, H), lambda i: (i, 0))] + [full(a) for a in small],
        out_specs=pl.BlockSpec((H, NODE_BLOCK), lambda i: (0, i)),
        out_shape=jax.ShapeDtypeStruct((H, N_PAD), jnp.float32),
    )(xnt, x_agg, *small)


HALF = N // 2
ACC_ROWS = 25056
TRASH = HALF
C = 80
CHUNKS = (E // 16) // C
NBUF = 5
MAIN = (CHUNKS // NBUF) * NBUF
ZROWS = 54
ZREP = 29
OUT_PER_TILE = 1560


def _sc_scatter(x_edge1, src_idx, dst_idx):
    mesh = plsc.VectorSubcoreMesh(core_axis_name="c", subcore_axis_name="s",
                                  num_cores=2, num_subcores=16)

    @functools.partial(
        pl.kernel,
        out_type=jax.ShapeDtypeStruct((N_PAD, H), jnp.float32),
        mesh=mesh,
        compiler_params=pltpu.CompilerParams(use_tc_tiling_on_sc=False),
        scratch_types=[
            pltpu.VMEM_SHARED((ACC_ROWS, H), jnp.float32),
            [pltpu.VMEM((C, H), jnp.float32)] * NBUF,
            [pltpu.VMEM((C,), jnp.int32)] * NBUF,
            [pltpu.VMEM((C,), jnp.int32)] * NBUF,
            pltpu.VMEM((ZROWS, H), jnp.float32),
            [pltpu.SemaphoreType.DMA] * NBUF,
            [pltpu.SemaphoreType.DMA] * NBUF,
        ],
    )
    def scatter_kernel(xe_hbm, src_hbm, dst_hbm, out_hbm, acc_sh, rows_v, src_v, dst_v, zbuf,
                       load_sem, scat_sem):
        c = lax.axis_index("c")
        s = lax.axis_index("s")
        z16 = jnp.zeros((16,), jnp.float32)

        def zero_zbuf(i, carry):
            for j in range(H // 16):
                zbuf[i, pl.ds(j * 16, 16)] = z16
            return carry
        lax.fori_loop(0, ZROWS, zero_zbuf, 0)

        def zero_acc(k, carry):
            pltpu.sync_copy(zbuf, acc_sh.at[pl.ds(s * (ZROWS * ZREP) + k * ZROWS, ZROWS)])
            return carry
        lax.fori_loop(0, ZREP, zero_acc, 0)
        plsc.subcore_barrier()

        lo = c * HALF
        tile_base = s * (C * CHUNKS)

        def clamp(buf):
            for g in range(C // 16):
                v = buf[pl.ds(g * 16, 16)] - lo
                buf[pl.ds(g * 16, 16)] = jnp.where((v >= 0) & (v < HALF), v, TRASH)

        def issue_loads(k, b):
            base = tile_base + k * C
            pltpu.async_copy(xe_hbm.at[pl.ds(base, C), pl.ds(0, H)], rows_v[b], load_sem[b])
            pltpu.async_copy(src_hbm.at[pl.ds(base, C)], src_v[b], load_sem[b])
            pltpu.async_copy(dst_hbm.at[pl.ds(base, C)], dst_v[b], load_sem[b])

        def drain_loads(b):
            pltpu.make_async_copy(xe_hbm.at[pl.ds(0, C), pl.ds(0, H)], rows_v[b], load_sem[b]).wait()
            pltpu.make_async_copy(src_hbm.at[pl.ds(0, C)], src_v[b], load_sem[b]).wait()
            pltpu.make_async_copy(dst_hbm.at[pl.ds(0, C)], dst_v[b], load_sem[b]).wait()

        def drain_scats(b):
            pltpu.make_async_copy(rows_v[b], acc_sh.at[src_v[b]], scat_sem[b]).wait()
            pltpu.make_async_copy(rows_v[b], acc_sh.at[dst_v[b]], scat_sem[b]).wait()

        issue_loads(0, 0)
        issue_loads(1, 1)

        def outer_body(t, carry):
            for b in range(NBUF):
                k = t * NBUF + b
                nb = (b + 2) % NBUF

                @pl.when(k >= NBUF - 2)
                def _drain():
                    drain_scats(nb)

                @pl.when(k + 2 < MAIN)
                def _prefetch():
                    issue_loads(k + 2, nb)

                drain_loads(b)
                clamp(src_v[b])
                clamp(dst_v[b])
                pltpu.async_copy(rows_v[b], acc_sh.at[src_v[b]], scat_sem[b], add=True)
                pltpu.async_copy(rows_v[b], acc_sh.at[dst_v[b]], scat_sem[b], add=True)
            return carry
        lax.fori_loop(0, MAIN // NBUF, outer_body, 0)
        for j in range(MAIN - (NBUF - 2), MAIN):
            drain_scats(j % NBUF)

        for k in range(MAIN, CHUNKS):
            base = tile_base + k * C
            pltpu.sync_copy(src_hbm.at[pl.ds(base, C)], src_v[0])
            pltpu.sync_copy(dst_hbm.at[pl.ds(base, C)], dst_v[0])
            pltpu.sync_copy(xe_hbm.at[pl.ds(base, C), pl.ds(0, H)], rows_v[0])
            clamp(src_v[0])
            clamp(dst_v[0])
            pltpu.sync_copy(rows_v[0], acc_sh.at[src_v[0]], add=True)
            pltpu.sync_copy(rows_v[0], acc_sh.at[dst_v[0]], add=True)
        plsc.subcore_barrier()

        pltpu.sync_copy(acc_sh.at[pl.ds(s * OUT_PER_TILE, OUT_PER_TILE)],
                        out_hbm.at[pl.ds(c * HALF + s * OUT_PER_TILE, OUT_PER_TILE)])

        @pl.when(s == 0)
        def _copy_tail():
            pltpu.sync_copy(acc_sh.at[pl.ds(16 * OUT_PER_TILE, HALF - 16 * OUT_PER_TILE)],
                            out_hbm.at[pl.ds(c * HALF + 16 * OUT_PER_TILE,
                                             HALF - 16 * OUT_PER_TILE)])

    return scatter_kernel(x_edge1, src_idx, dst_idx)


def kernel(x_node, x_edge, edge_index, p0, p1):
    x_edge1_t, x_edge1_packed = _edge_mlp(x_edge, p0)
    x_agg = _sc_scatter(x_edge1_packed, edge_index[:, 0], edge_index[:, 1])
    x_node3_t = _node_mlp(x_node, x_agg, p1)
    return (x_node3_t[:, :N].T, x_edge1_t.T)


# speedup vs baseline: 5.4291x; 1.7550x over previous
import functools

import jax
import jax.numpy as jnp
from jax import lax
from jax.experimental import pallas as pl
from jax.experimental.pallas import tpu as pltpu
from jax.experimental.pallas import tpu_sc as plsc

N = 50000
E = 800000
DE = 32
DN = 32
H = 64

EDGE_BLOCK = 3200
N_PAD = 50048
NODE_BLOCK = 2176


def _bf(x):
    return x.astype(jnp.bfloat16)


def _layer_norm_t(h, g, beta):
    mu = jnp.mean(h, axis=0, keepdims=True)
    var = jnp.mean((h - mu) ** 2, axis=0, keepdims=True)
    return (h - mu) * lax.rsqrt(var + 1e-5) * g + beta


def _edge_mlp_body(xt_ref, w1, b1, w2, b2, w3, b3, w4, b4, g, beta, outt_ref, packed_ref):
    xt = xt_ref[...]
    h = jnp.maximum(jnp.dot(_bf(w1[...]), _bf(xt), preferred_element_type=jnp.float32) + b1[...], 0.0)
    h = jnp.maximum(jnp.dot(_bf(w2[...]), _bf(h), preferred_element_type=jnp.float32) + b2[...], 0.0)
    h = jnp.maximum(jnp.dot(_bf(w3[...]), _bf(h), preferred_element_type=jnp.float32) + b3[...], 0.0)
    h = jnp.dot(_bf(w4[...]), _bf(h), preferred_element_type=jnp.float32) + b4[...]
    out = _layer_norm_t(h, g[...], beta[...])
    outt_ref[...] = out
    packed_ref[:, pl.ds(0, H)] = out.T


def _edge_mlp(x_edge, p0):
    W1, b1, W2, b2, W3, b3, W4, b4, g, beta = p0
    small = [W1.T, b1.reshape(-1, 1), W2.T, b2.reshape(-1, 1), W3.T, b3.reshape(-1, 1),
             W4.T, b4.reshape(-1, 1), g.reshape(-1, 1), beta.reshape(-1, 1)]
    grid = (E // EDGE_BLOCK,)
    full = lambda a: pl.BlockSpec(a.shape, lambda i: (0, 0))
    return pl.pallas_call(
        _edge_mlp_body,
        grid=grid,
        in_specs=[pl.BlockSpec((DE, EDGE_BLOCK), lambda i: (0, i))] + [full(a) for a in small],
        out_specs=[pl.BlockSpec((H, EDGE_BLOCK), lambda i: (0, i)),
                   pl.BlockSpec((EDGE_BLOCK, 2 * H), lambda i: (i, 0))],
        out_shape=[jax.ShapeDtypeStruct((H, E), jnp.float32),
                   jax.ShapeDtypeStruct((E, 2 * H), jnp.float32)],
    )(x_edge.T, *small)


def _node_mlp_body(xnt_ref, xa_ref, w1a, w1b, b1, w2, b2, w3, b3, w4, b4, g, beta, outt_ref):
    xat = xa_ref[...].T
    h = (jnp.dot(w1a[...], xnt_ref[...], preferred_element_type=jnp.float32)
         + jnp.dot(w1b[...], xat, preferred_element_type=jnp.float32) + b1[...])
    h = jnp.maximum(h, 0.0)
    h = jnp.maximum(jnp.dot(w2[...], h, preferred_element_type=jnp.float32) + b2[...], 0.0)
    h = jnp.maximum(jnp.dot(w3[...], h, preferred_element_type=jnp.float32) + b3[...], 0.0)
    h = jnp.dot(w4[...], h, preferred_element_type=jnp.float32) + b4[...]
    outt_ref[...] = _layer_norm_t(h, g[...], beta[...])


def _node_mlp(x_node, x_agg, p1):
    W1, b1, W2, b2, W3, b3, W4, b4, g, beta = p1
    small = [W1[:DN].T, W1[DN:].T, b1.reshape(-1, 1), W2.T, b2.reshape(-1, 1), W3.T,
             b3.reshape(-1, 1), W4.T, b4.reshape(-1, 1), g.reshape(-1, 1), beta.reshape(-1, 1)]
    grid = (N_PAD // NODE_BLOCK,)
    full = lambda a: pl.BlockSpec(a.shape, lambda i: (0, 0))
    xnt = jnp.pad(x_node.T, ((0, 0), (0, N_PAD - N)))
    return pl.pallas_call(
        _node_mlp_body,
        grid=grid,
        in_specs=[pl.BlockSpec((DN, NODE_BLOCK), lambda i: (0, i)),
                  pl.BlockSpec((NODE_BLOCK, H), lambda i: (i, 0))] + [full(a) for a in small],
        out_specs=pl.BlockSpec((H, NODE_BLOCK), lambda i: (0, i)),
        out_shape=jax.ShapeDtypeStruct((H, N_PAD), jnp.float32),
    )(xnt, x_agg, *small)


HALF = N // 2
ACC_ROWS = 25056
TRASH = HALF
C = 80
CHUNKS = (E // 16) // C
NBUF = 5
MAIN = (CHUNKS // NBUF) * NBUF
ZROWS = 54
ZREP = 29
OUT_PER_TILE = 1560


def _sc_scatter(x_edge1, src_idx, dst_idx):
    mesh = plsc.VectorSubcoreMesh(core_axis_name="c", subcore_axis_name="s",
                                  num_cores=2, num_subcores=16)

    @functools.partial(
        pl.kernel,
        out_type=jax.ShapeDtypeStruct((N_PAD, H), jnp.float32),
        mesh=mesh,
        compiler_params=pltpu.CompilerParams(use_tc_tiling_on_sc=False),
        scratch_types=[
            pltpu.VMEM_SHARED((ACC_ROWS, H), jnp.float32),
            [pltpu.VMEM((C, H), jnp.float32)] * NBUF,
            [pltpu.VMEM((C,), jnp.int32)] * NBUF,
            [pltpu.VMEM((C,), jnp.int32)] * NBUF,
            pltpu.VMEM((ZROWS, H), jnp.float32),
            [pltpu.SemaphoreType.DMA] * NBUF,
            [pltpu.SemaphoreType.DMA] * NBUF,
        ],
    )
    def scatter_kernel(xe_hbm, src_hbm, dst_hbm, out_hbm, acc_sh, rows_v, src_v, dst_v, zbuf,
                       load_sem, scat_sem):
        c = lax.axis_index("c")
        s = lax.axis_index("s")
        z16 = jnp.zeros((16,), jnp.float32)

        def zero_zbuf(i, carry):
            for j in range(H // 16):
                zbuf[i, pl.ds(j * 16, 16)] = z16
            return carry
        lax.fori_loop(0, ZROWS, zero_zbuf, 0)

        def zero_acc(k, carry):
            pltpu.sync_copy(zbuf, acc_sh.at[pl.ds(s * (ZROWS * ZREP) + k * ZROWS, ZROWS)])
            return carry
        lax.fori_loop(0, ZREP, zero_acc, 0)
        plsc.subcore_barrier()

        lo = c * HALF
        tile_base = s * (C * CHUNKS)

        def clamp(buf):
            for g in range(C // 16):
                v = buf[pl.ds(g * 16, 16)] - lo
                buf[pl.ds(g * 16, 16)] = jnp.where((v >= 0) & (v < HALF), v,
                                                   TRASH + (v & 31))

        def issue_loads(k, b):
            base = tile_base + k * C
            pltpu.async_copy(xe_hbm.at[pl.ds(base, C), pl.ds(0, H)], rows_v[b], load_sem[b])
            pltpu.async_copy(src_hbm.at[pl.ds(base, C)], src_v[b], load_sem[b])
            pltpu.async_copy(dst_hbm.at[pl.ds(base, C)], dst_v[b], load_sem[b])

        def drain_loads(b):
            pltpu.make_async_copy(xe_hbm.at[pl.ds(0, C), pl.ds(0, H)], rows_v[b], load_sem[b]).wait()
            pltpu.make_async_copy(src_hbm.at[pl.ds(0, C)], src_v[b], load_sem[b]).wait()
            pltpu.make_async_copy(dst_hbm.at[pl.ds(0, C)], dst_v[b], load_sem[b]).wait()

        def drain_scats(b):
            pltpu.make_async_copy(rows_v[b], acc_sh.at[src_v[b]], scat_sem[b]).wait()
            pltpu.make_async_copy(rows_v[b], acc_sh.at[dst_v[b]], scat_sem[b]).wait()

        issue_loads(0, 0)
        issue_loads(1, 1)

        def outer_body(t, carry):
            for b in range(NBUF):
                k = t * NBUF + b
                nb = (b + 2) % NBUF

                @pl.when(k >= NBUF - 2)
                def _drain():
                    drain_scats(nb)

                @pl.when(k + 2 < MAIN)
                def _prefetch():
                    issue_loads(k + 2, nb)

                drain_loads(b)
                clamp(src_v[b])
                clamp(dst_v[b])
                pltpu.async_copy(rows_v[b], acc_sh.at[src_v[b]], scat_sem[b], add=True)
                pltpu.async_copy(rows_v[b], acc_sh.at[dst_v[b]], scat_sem[b], add=True)
            return carry
        lax.fori_loop(0, MAIN // NBUF, outer_body, 0)
        for j in range(MAIN - (NBUF - 2), MAIN):
            drain_scats(j % NBUF)

        for k in range(MAIN, CHUNKS):
            base = tile_base + k * C
            pltpu.sync_copy(src_hbm.at[pl.ds(base, C)], src_v[0])
            pltpu.sync_copy(dst_hbm.at[pl.ds(base, C)], dst_v[0])
            pltpu.sync_copy(xe_hbm.at[pl.ds(base, C), pl.ds(0, H)], rows_v[0])
            clamp(src_v[0])
            clamp(dst_v[0])
            pltpu.sync_copy(rows_v[0], acc_sh.at[src_v[0]], add=True)
            pltpu.sync_copy(rows_v[0], acc_sh.at[dst_v[0]], add=True)
        plsc.subcore_barrier()

        pltpu.sync_copy(acc_sh.at[pl.ds(s * OUT_PER_TILE, OUT_PER_TILE)],
                        out_hbm.at[pl.ds(c * HALF + s * OUT_PER_TILE, OUT_PER_TILE)])

        @pl.when(s == 0)
        def _copy_tail():
            pltpu.sync_copy(acc_sh.at[pl.ds(16 * OUT_PER_TILE, HALF - 16 * OUT_PER_TILE)],
                            out_hbm.at[pl.ds(c * HALF + 16 * OUT_PER_TILE,
                                             HALF - 16 * OUT_PER_TILE)])

    return scatter_kernel(x_edge1, src_idx, dst_idx)


def kernel(x_node, x_edge, edge_index, p0, p1):
    x_edge1_t, x_edge1_packed = _edge_mlp(x_edge, p0)
    x_agg = _sc_scatter(x_edge1_packed, edge_index[:, 0], edge_index[:, 1])
    x_node3_t = _node_mlp(x_node, x_agg, p1)
    return (x_node3_t[:, :N].T, x_edge1_t.T)


# speedup vs baseline: 5.7624x; 1.0614x over previous
import functools

import jax
import jax.numpy as jnp
from jax import lax
from jax.experimental import pallas as pl
from jax.experimental.pallas import tpu as pltpu
from jax.experimental.pallas import tpu_sc as plsc

N = 50000
E = 800000
DE = 32
DN = 32
H = 64

EDGE_BLOCK = 3200
N_PAD = 50048
NODE_BLOCK = 2176


def _bf(x):
    return x.astype(jnp.bfloat16)


def _layer_norm_t(h, g, beta):
    mu = jnp.mean(h, axis=0, keepdims=True)
    var = jnp.mean((h - mu) ** 2, axis=0, keepdims=True)
    return (h - mu) * lax.rsqrt(var + 1e-5) * g + beta


def _edge_mlp_body(xt_ref, w1, b1, w2, b2, w3, b3, w4, b4, g, beta, packed_ref):
    xt = xt_ref[...]
    h = jnp.maximum(jnp.dot(_bf(w1[...]), _bf(xt), preferred_element_type=jnp.float32) + b1[...], 0.0)
    h = jnp.maximum(jnp.dot(_bf(w2[...]), _bf(h), preferred_element_type=jnp.float32) + b2[...], 0.0)
    h = jnp.maximum(jnp.dot(_bf(w3[...]), _bf(h), preferred_element_type=jnp.float32) + b3[...], 0.0)
    h = jnp.dot(_bf(w4[...]), _bf(h), preferred_element_type=jnp.float32) + b4[...]
    out = _layer_norm_t(h, g[...], beta[...])
    packed_ref[:, pl.ds(0, H)] = out.T


def _edge_mlp(x_edge, p0):
    W1, b1, W2, b2, W3, b3, W4, b4, g, beta = p0
    small = [W1.T, b1.reshape(-1, 1), W2.T, b2.reshape(-1, 1), W3.T, b3.reshape(-1, 1),
             W4.T, b4.reshape(-1, 1), g.reshape(-1, 1), beta.reshape(-1, 1)]
    grid = (E // EDGE_BLOCK,)
    full = lambda a: pl.BlockSpec(a.shape, lambda i: (0, 0))
    return pl.pallas_call(
        _edge_mlp_body,
        grid=grid,
        in_specs=[pl.BlockSpec((DE, EDGE_BLOCK), lambda i: (0, i))] + [full(a) for a in small],
        out_specs=pl.BlockSpec((EDGE_BLOCK, 2 * H), lambda i: (i, 0)),
        out_shape=jax.ShapeDtypeStruct((E, 2 * H), jnp.float32),
    )(x_edge.T, *small)


def _transpose_packed(packed):
    def body(p_ref, outt_ref):
        outt_ref[...] = p_ref[:, pl.ds(0, H)].T

    return pl.pallas_call(
        body,
        grid=(E // EDGE_BLOCK,),
        in_specs=[pl.BlockSpec((EDGE_BLOCK, 2 * H), lambda i: (i, 0))],
        out_specs=pl.BlockSpec((H, EDGE_BLOCK), lambda i: (0, i)),
        out_shape=jax.ShapeDtypeStruct((H, E), jnp.float32),
    )(packed)


def _node_mlp_body(xnt_ref, xa_ref, w1a, w1b, b1, w2, b2, w3, b3, w4, b4, g, beta, outt_ref):
    xat = xa_ref[...].T
    h = (jnp.dot(w1a[...], xnt_ref[...], preferred_element_type=jnp.float32)
         + jnp.dot(w1b[...], xat, preferred_element_type=jnp.float32) + b1[...])
    h = jnp.maximum(h, 0.0)
    h = jnp.maximum(jnp.dot(w2[...], h, preferred_element_type=jnp.float32) + b2[...], 0.0)
    h = jnp.maximum(jnp.dot(w3[...], h, preferred_element_type=jnp.float32) + b3[...], 0.0)
    h = jnp.dot(w4[...], h, preferred_element_type=jnp.float32) + b4[...]
    outt_ref[...] = _layer_norm_t(h, g[...], beta[...])


def _node_mlp(x_node, x_agg, p1):
    W1, b1, W2, b2, W3, b3, W4, b4, g, beta = p1
    small = [W1[:DN].T, W1[DN:].T, b1.reshape(-1, 1), W2.T, b2.reshape(-1, 1), W3.T,
             b3.reshape(-1, 1), W4.T, b4.reshape(-1, 1), g.reshape(-1, 1), beta.reshape(-1, 1)]
    grid = (N_PAD // NODE_BLOCK,)
    full = lambda a: pl.BlockSpec(a.shape, lambda i: (0, 0))
    xnt = jnp.pad(x_node.T, ((0, 0), (0, N_PAD - N)))
    return pl.pallas_call(
        _node_mlp_body,
        grid=grid,
        in_specs=[pl.BlockSpec((DN, NODE_BLOCK), lambda i: (0, i)),
                  pl.BlockSpec((NODE_BLOCK, H), lambda i: (i, 0))] + [full(a) for a in small],
        out_specs=pl.BlockSpec((H, NODE_BLOCK), lambda i: (0, i)),
        out_shape=jax.ShapeDtypeStruct((H, N_PAD), jnp.float32),
    )(xnt, x_agg, *small)


HALF = N // 2
ACC_ROWS = 25056
TRASH = HALF
C = 80
CHUNKS = (E // 16) // C
NBUF = 5
MAIN = (CHUNKS // NBUF) * NBUF
ZROWS = 54
ZREP = 29
OUT_PER_TILE = 1560


def _sc_scatter(x_edge1, eidxt):
    mesh = plsc.VectorSubcoreMesh(core_axis_name="c", subcore_axis_name="s",
                                  num_cores=2, num_subcores=16)

    @functools.partial(
        pl.kernel,
        out_type=jax.ShapeDtypeStruct((N_PAD, H), jnp.float32),
        mesh=mesh,
        compiler_params=pltpu.CompilerParams(use_tc_tiling_on_sc=False),
        scratch_types=[
            pltpu.VMEM_SHARED((ACC_ROWS, H), jnp.float32),
            [pltpu.VMEM((C, H), jnp.float32)] * NBUF,
            [pltpu.VMEM((C,), jnp.int32)] * NBUF,
            [pltpu.VMEM((C,), jnp.int32)] * NBUF,
            pltpu.VMEM((ZROWS, H), jnp.float32),
            [pltpu.SemaphoreType.DMA] * NBUF,
            [pltpu.SemaphoreType.DMA] * NBUF,
        ],
    )
    def scatter_kernel(xe_hbm, eidxt_hbm, out_hbm, acc_sh, rows_v, src_v, dst_v, zbuf,
                       load_sem, scat_sem):
        c = lax.axis_index("c")
        s = lax.axis_index("s")
        z16 = jnp.zeros((16,), jnp.float32)

        def zero_zbuf(i, carry):
            for j in range(H // 16):
                zbuf[i, pl.ds(j * 16, 16)] = z16
            return carry
        lax.fori_loop(0, ZROWS, zero_zbuf, 0)

        def zero_acc(k, carry):
            pltpu.sync_copy(zbuf, acc_sh.at[pl.ds(s * (ZROWS * ZREP) + k * ZROWS, ZROWS)])
            return carry
        lax.fori_loop(0, ZREP, zero_acc, 0)
        plsc.subcore_barrier()

        lo = c * HALF
        tile_base = s * (C * CHUNKS)

        def clamp(buf):
            for g in range(C // 16):
                v = buf[pl.ds(g * 16, 16)] - lo
                buf[pl.ds(g * 16, 16)] = jnp.where((v >= 0) & (v < HALF), v,
                                                   TRASH + (v & 31))

        def issue_loads(k, b):
            base = tile_base + k * C
            pltpu.async_copy(xe_hbm.at[pl.ds(base, C), pl.ds(0, H)], rows_v[b], load_sem[b])
            pltpu.async_copy(eidxt_hbm.at[0, pl.ds(base, C)], src_v[b], load_sem[b])
            pltpu.async_copy(eidxt_hbm.at[1, pl.ds(base, C)], dst_v[b], load_sem[b])

        def drain_loads(b):
            pltpu.make_async_copy(xe_hbm.at[pl.ds(0, C), pl.ds(0, H)], rows_v[b], load_sem[b]).wait()
            pltpu.make_async_copy(eidxt_hbm.at[0, pl.ds(0, C)], src_v[b], load_sem[b]).wait()
            pltpu.make_async_copy(eidxt_hbm.at[1, pl.ds(0, C)], dst_v[b], load_sem[b]).wait()

        def drain_scats(b):
            pltpu.make_async_copy(rows_v[b], acc_sh.at[src_v[b]], scat_sem[b]).wait()
            pltpu.make_async_copy(rows_v[b], acc_sh.at[dst_v[b]], scat_sem[b]).wait()

        issue_loads(0, 0)
        issue_loads(1, 1)

        def outer_body(t, carry):
            for b in range(NBUF):
                k = t * NBUF + b
                nb = (b + 2) % NBUF

                @pl.when(k >= NBUF - 2)
                def _drain():
                    drain_scats(nb)

                @pl.when(k + 2 < MAIN)
                def _prefetch():
                    issue_loads(k + 2, nb)

                drain_loads(b)
                clamp(src_v[b])
                clamp(dst_v[b])
                pltpu.async_copy(rows_v[b], acc_sh.at[src_v[b]], scat_sem[b], add=True)
                pltpu.async_copy(rows_v[b], acc_sh.at[dst_v[b]], scat_sem[b], add=True)
            return carry
        lax.fori_loop(0, MAIN // NBUF, outer_body, 0)
        for j in range(MAIN - (NBUF - 2), MAIN):
            drain_scats(j % NBUF)

        for k in range(MAIN, CHUNKS):
            base = tile_base + k * C
            pltpu.sync_copy(eidxt_hbm.at[0, pl.ds(base, C)], src_v[0])
            pltpu.sync_copy(eidxt_hbm.at[1, pl.ds(base, C)], dst_v[0])
            pltpu.sync_copy(xe_hbm.at[pl.ds(base, C), pl.ds(0, H)], rows_v[0])
            clamp(src_v[0])
            clamp(dst_v[0])
            pltpu.sync_copy(rows_v[0], acc_sh.at[src_v[0]], add=True)
            pltpu.sync_copy(rows_v[0], acc_sh.at[dst_v[0]], add=True)
        plsc.subcore_barrier()

        pltpu.sync_copy(acc_sh.at[pl.ds(s * OUT_PER_TILE, OUT_PER_TILE)],
                        out_hbm.at[pl.ds(c * HALF + s * OUT_PER_TILE, OUT_PER_TILE)])

        @pl.when(s == 0)
        def _copy_tail():
            pltpu.sync_copy(acc_sh.at[pl.ds(16 * OUT_PER_TILE, HALF - 16 * OUT_PER_TILE)],
                            out_hbm.at[pl.ds(c * HALF + 16 * OUT_PER_TILE,
                                             HALF - 16 * OUT_PER_TILE)])

    return scatter_kernel(x_edge1, eidxt)


def kernel(x_node, x_edge, edge_index, p0, p1):
    x_edge1_packed = _edge_mlp(x_edge, p0)
    x_agg = _sc_scatter(x_edge1_packed, edge_index.T)
    x_edge1_t = _transpose_packed(x_edge1_packed)
    x_node3_t = _node_mlp(x_node, x_agg, p1)
    return (x_node3_t[:, :N].T, x_edge1_t.T)


# speedup vs baseline: 6.2893x; 1.0915x over previous
import functools

import jax
import jax.numpy as jnp
from jax import lax
from jax.experimental import pallas as pl
from jax.experimental.pallas import tpu as pltpu
from jax.experimental.pallas import tpu_sc as plsc

N = 50000
E = 800000
DE = 32
DN = 32
H = 64

EDGE_BLOCK = 6400
N_PAD = 50048
NODE_BLOCK = 2176


def _bf(x):
    return x.astype(jnp.bfloat16)


def _layer_norm_t(h, g, beta):
    mu = jnp.mean(h, axis=0, keepdims=True)
    var = jnp.mean((h - mu) ** 2, axis=0, keepdims=True)
    return (h - mu) * lax.rsqrt(var + 1e-5) * g + beta


def _edge_mlp_body(xt_ref, w1, b1, w2, b2, w3, b3, w4, b4, g, beta, packed_ref):
    xt = xt_ref[...]
    h = jnp.maximum(jnp.dot(_bf(w1[...]), _bf(xt), preferred_element_type=jnp.float32) + b1[...], 0.0)
    h = jnp.maximum(jnp.dot(_bf(w2[...]), _bf(h), preferred_element_type=jnp.float32) + b2[...], 0.0)
    h = jnp.maximum(jnp.dot(_bf(w3[...]), _bf(h), preferred_element_type=jnp.float32) + b3[...], 0.0)
    h = jnp.dot(_bf(w4[...]), _bf(h), preferred_element_type=jnp.float32) + b4[...]
    out = _layer_norm_t(h, g[...], beta[...])
    packed_ref[:, pl.ds(0, H)] = out.T


def _edge_mlp(x_edge, p0):
    W1, b1, W2, b2, W3, b3, W4, b4, g, beta = p0
    small = [W1.T, b1.reshape(-1, 1), W2.T, b2.reshape(-1, 1), W3.T, b3.reshape(-1, 1),
             W4.T, b4.reshape(-1, 1), g.reshape(-1, 1), beta.reshape(-1, 1)]
    grid = (E // EDGE_BLOCK,)
    full = lambda a: pl.BlockSpec(a.shape, lambda i: (0, 0))
    return pl.pallas_call(
        _edge_mlp_body,
        grid=grid,
        in_specs=[pl.BlockSpec((DE, EDGE_BLOCK), lambda i: (0, i))] + [full(a) for a in small],
        out_specs=pl.BlockSpec((EDGE_BLOCK, 2 * H), lambda i: (i, 0)),
        out_shape=jax.ShapeDtypeStruct((E, 2 * H), jnp.float32),
    )(x_edge.T, *small)


def _transpose_packed(packed):
    def body(p_ref, outt_ref):
        outt_ref[...] = p_ref[:, pl.ds(0, H)].T

    return pl.pallas_call(
        body,
        grid=(E // EDGE_BLOCK,),
        in_specs=[pl.BlockSpec((EDGE_BLOCK, 2 * H), lambda i: (i, 0))],
        out_specs=pl.BlockSpec((H, EDGE_BLOCK), lambda i: (0, i)),
        out_shape=jax.ShapeDtypeStruct((H, E), jnp.float32),
    )(packed)


def _node_mlp_body(xnt_ref, xa_ref, w1a, w1b, b1, w2, b2, w3, b3, w4, b4, g, beta, outt_ref):
    xat = xa_ref[...].T
    h = (jnp.dot(w1a[...], xnt_ref[...], preferred_element_type=jnp.float32)
         + jnp.dot(w1b[...], xat, preferred_element_type=jnp.float32) + b1[...])
    h = jnp.maximum(h, 0.0)
    h = jnp.maximum(jnp.dot(w2[...], h, preferred_element_type=jnp.float32) + b2[...], 0.0)
    h = jnp.maximum(jnp.dot(w3[...], h, preferred_element_type=jnp.float32) + b3[...], 0.0)
    h = jnp.dot(w4[...], h, preferred_element_type=jnp.float32) + b4[...]
    outt_ref[...] = _layer_norm_t(h, g[...], beta[...])


def _node_mlp(x_node, x_agg, p1):
    W1, b1, W2, b2, W3, b3, W4, b4, g, beta = p1
    small = [W1[:DN].T, W1[DN:].T, b1.reshape(-1, 1), W2.T, b2.reshape(-1, 1), W3.T,
             b3.reshape(-1, 1), W4.T, b4.reshape(-1, 1), g.reshape(-1, 1), beta.reshape(-1, 1)]
    grid = (N_PAD // NODE_BLOCK,)
    full = lambda a: pl.BlockSpec(a.shape, lambda i: (0, 0))
    xnt = jnp.pad(x_node.T, ((0, 0), (0, N_PAD - N)))
    return pl.pallas_call(
        _node_mlp_body,
        grid=grid,
        in_specs=[pl.BlockSpec((DN, NODE_BLOCK), lambda i: (0, i)),
                  pl.BlockSpec((NODE_BLOCK, H), lambda i: (i, 0))] + [full(a) for a in small],
        out_specs=pl.BlockSpec((H, NODE_BLOCK), lambda i: (0, i)),
        out_shape=jax.ShapeDtypeStruct((H, N_PAD), jnp.float32),
    )(xnt, x_agg, *small)


HALF = N // 2
ACC_ROWS = 25056
TRASH = HALF
C = 80
CHUNKS = (E // 16) // C
NBUF = 5
MAIN = (CHUNKS // NBUF) * NBUF
ZROWS = 54
ZREP = 29
OUT_PER_TILE = 1560


def _sc_scatter(x_edge1, eidxt):
    mesh = plsc.VectorSubcoreMesh(core_axis_name="c", subcore_axis_name="s",
                                  num_cores=2, num_subcores=16)

    @functools.partial(
        pl.kernel,
        out_type=jax.ShapeDtypeStruct((N_PAD, H), jnp.float32),
        mesh=mesh,
        compiler_params=pltpu.CompilerParams(use_tc_tiling_on_sc=False),
        scratch_types=[
            pltpu.VMEM_SHARED((ACC_ROWS, H), jnp.float32),
            [pltpu.VMEM((C, H), jnp.float32)] * NBUF,
            [pltpu.VMEM((C,), jnp.int32)] * NBUF,
            [pltpu.VMEM((C,), jnp.int32)] * NBUF,
            pltpu.VMEM((ZROWS, H), jnp.float32),
            [pltpu.SemaphoreType.DMA] * NBUF,
            [pltpu.SemaphoreType.DMA] * NBUF,
        ],
    )
    def scatter_kernel(xe_hbm, eidxt_hbm, out_hbm, acc_sh, rows_v, src_v, dst_v, zbuf,
                       load_sem, scat_sem):
        c = lax.axis_index("c")
        s = lax.axis_index("s")
        z16 = jnp.zeros((16,), jnp.float32)

        def zero_zbuf(i, carry):
            for j in range(H // 16):
                zbuf[i, pl.ds(j * 16, 16)] = z16
            return carry
        lax.fori_loop(0, ZROWS, zero_zbuf, 0)

        def zero_acc(k, carry):
            pltpu.sync_copy(zbuf, acc_sh.at[pl.ds(s * (ZROWS * ZREP) + k * ZROWS, ZROWS)])
            return carry
        lax.fori_loop(0, ZREP, zero_acc, 0)
        plsc.subcore_barrier()

        lo = c * HALF
        tile_base = s * (C * CHUNKS)

        def clamp(buf):
            for g in range(C // 16):
                v = buf[pl.ds(g * 16, 16)] - lo
                buf[pl.ds(g * 16, 16)] = jnp.where((v >= 0) & (v < HALF), v,
                                                   TRASH + (v & 31) + s)

        def issue_loads(k, b):
            base = tile_base + k * C
            pltpu.async_copy(xe_hbm.at[pl.ds(base, C), pl.ds(0, H)], rows_v[b], load_sem[b])
            pltpu.async_copy(eidxt_hbm.at[0, pl.ds(base, C)], src_v[b], load_sem[b])
            pltpu.async_copy(eidxt_hbm.at[1, pl.ds(base, C)], dst_v[b], load_sem[b])

        def drain_loads(b):
            pltpu.make_async_copy(xe_hbm.at[pl.ds(0, C), pl.ds(0, H)], rows_v[b], load_sem[b]).wait()
            pltpu.make_async_copy(eidxt_hbm.at[0, pl.ds(0, C)], src_v[b], load_sem[b]).wait()
            pltpu.make_async_copy(eidxt_hbm.at[1, pl.ds(0, C)], dst_v[b], load_sem[b]).wait()

        def drain_scats(b):
            pltpu.make_async_copy(rows_v[b], acc_sh.at[src_v[b]], scat_sem[b]).wait()
            pltpu.make_async_copy(rows_v[b], acc_sh.at[dst_v[b]], scat_sem[b]).wait()

        issue_loads(0, 0)
        issue_loads(1, 1)

        def outer_body(t, carry):
            for b in range(NBUF):
                k = t * NBUF + b
                nb = (b + 2) % NBUF

                @pl.when(k >= NBUF - 2)
                def _drain():
                    drain_scats(nb)

                @pl.when(k + 2 < MAIN)
                def _prefetch():
                    issue_loads(k + 2, nb)

                drain_loads(b)
                clamp(src_v[b])
                clamp(dst_v[b])
                pltpu.async_copy(rows_v[b], acc_sh.at[src_v[b]], scat_sem[b], add=True)
                pltpu.async_copy(rows_v[b], acc_sh.at[dst_v[b]], scat_sem[b], add=True)
            return carry
        lax.fori_loop(0, MAIN // NBUF, outer_body, 0)
        for j in range(MAIN - (NBUF - 2), MAIN):
            drain_scats(j % NBUF)

        for k in range(MAIN, CHUNKS):
            base = tile_base + k * C
            pltpu.sync_copy(eidxt_hbm.at[0, pl.ds(base, C)], src_v[0])
            pltpu.sync_copy(eidxt_hbm.at[1, pl.ds(base, C)], dst_v[0])
            pltpu.sync_copy(xe_hbm.at[pl.ds(base, C), pl.ds(0, H)], rows_v[0])
            clamp(src_v[0])
            clamp(dst_v[0])
            pltpu.sync_copy(rows_v[0], acc_sh.at[src_v[0]], add=True)
            pltpu.sync_copy(rows_v[0], acc_sh.at[dst_v[0]], add=True)
        plsc.subcore_barrier()

        pltpu.sync_copy(acc_sh.at[pl.ds(s * OUT_PER_TILE, OUT_PER_TILE)],
                        out_hbm.at[pl.ds(c * HALF + s * OUT_PER_TILE, OUT_PER_TILE)])

        @pl.when(s == 0)
        def _copy_tail():
            pltpu.sync_copy(acc_sh.at[pl.ds(16 * OUT_PER_TILE, HALF - 16 * OUT_PER_TILE)],
                            out_hbm.at[pl.ds(c * HALF + 16 * OUT_PER_TILE,
                                             HALF - 16 * OUT_PER_TILE)])

    return scatter_kernel(x_edge1, eidxt)


def kernel(x_node, x_edge, edge_index, p0, p1):
    x_edge1_packed = _edge_mlp(x_edge, p0)
    x_agg = _sc_scatter(x_edge1_packed, edge_index.T)
    x_edge1_t = _transpose_packed(x_edge1_packed)
    x_node3_t = _node_mlp(x_node, x_agg, p1)
    return (x_node3_t[:, :N].T, x_edge1_t.T)
